# Initial kernel scaffold; baseline (speedup 1.0000x reference)
import jax
import jax.numpy as jnp
from jax.experimental import pallas as pl


def kernel(activation, hidden_state, sparsity_k, in_proj_w, in_proj_b, out_proj_w, out_proj_b, gru_w_ih, gru_w_hh, gru_b_ih, gru_b_hh, ln_g, ln_b, act_w1, act_b1, act_w2, act_b2):
    raise NotImplementedError("write your pallas kernel here")


# speedup vs baseline: 1.7314x; 1.7314x over previous
import math

import jax
import jax.numpy as jnp
from jax import lax
from jax.experimental import pallas as pl
from jax.experimental.pallas import tpu as pltpu

N = 4096
D = 256
H = 4
DH = D // H
D3 = 3 * D

BLK_A = 512
BLK_Q = 256
BLK_D = 512
KK = 256

_NEG_INF = float("-inf")


def _nt(a, b):
    return lax.dot_general(a, b, (((1,), (1,)), ((), ())),
                           preferred_element_type=jnp.float32)


def _proj_kernel(x_ref, w_in_ref, b_in_ref, w_hh_ref, b_hh_ref,
                 q_ref, k_ref, v_ref, gh_ref):
    xs = x_ref[...]
    qkv = _nt(xs, w_in_ref[...]) + b_in_ref[...]
    q_ref[...] = qkv[:, :D]
    k_ref[...] = qkv[:, D:2 * D]
    v_ref[...] = qkv[:, 2 * D:]
    gh_ref[...] = _nt(xs, w_hh_ref[...]) + b_hh_ref[...]


def _attn_kernel(act_key_ref, act_col_ref, x_ref, q_ref, k_ref, v_ref, gh_ref,
                 w_out_ref, b_out_ref, w_ih_ref, b_ih_ref,
                 ln_g_ref, ln_b_ref, w1_ref, b1_ref, w2_ref, b2_ref,
                 act_out_ref, hid_out_ref):
    xs = x_ref[...]
    q = q_ref[...]
    kk = k_ref[...]
    vv = v_ref[...]
    maskk = act_key_ref[...] > 0.01
    scale = 1.0 / math.sqrt(DH)
    nq = q.shape[0]
    outs = []
    for h in range(H):
        qh = q[:, h * DH:(h + 1) * DH]
        kh = kk[:, h * DH:(h + 1) * DH]
        vh = vv[:, h * DH:(h + 1) * DH]
        s_full = _nt(qh, kh) * scale
        s_full = jnp.where(maskk, s_full, _NEG_INF)
        KC = 1024
        m_old = jnp.full((nq, 1), _NEG_INF, jnp.float32)
        l_old = jnp.zeros((nq, 1), jnp.float32)
        o_old = jnp.zeros((nq, DH), jnp.float32)
        for j in range(N // KC):
            s = s_full[:, j * KC:(j + 1) * KC]
            mb = jnp.max(s, axis=1, keepdims=True)
            m_new = jnp.maximum(m_old, mb)
            corr = jnp.where(m_old == m_new, 0.0, m_old - m_new)
            e = jnp.exp(s - m_new)
            ec = jnp.exp(corr)
            l_new = ec * l_old + jnp.sum(e, axis=1, keepdims=True)
            acc = (ec * l_old) * o_old
            onum = jnp.dot(e, vh[j * KC:(j + 1) * KC, :],
                           preferred_element_type=jnp.float32) + acc
            o_old = onum * (1.0 / l_new)
            m_old, l_old = m_new, l_new
        outs.append(o_old)
    o = jnp.concatenate(outs, axis=1)
    attn = _nt(o, w_out_ref[...]) + b_out_ref[...]
    acts = act_col_ref[...]
    msg = attn * acts
    gi = _nt(msg, w_ih_ref[...]) + b_ih_ref[...]
    gh = gh_ref[...]
    r = jax.nn.sigmoid(gi[:, :D] + gh[:, :D])
    z = jax.nn.sigmoid(gi[:, D:2 * D] + gh[:, D:2 * D])
    n = jnp.tanh(gi[:, 2 * D:] + r * gh[:, 2 * D:])
    ns = (1.0 - z) * n + z * xs
    mu = jnp.mean(ns, axis=1, keepdims=True)
    var = jnp.mean((ns - mu) ** 2, axis=1, keepdims=True)
    ns = (ns - mu) / jnp.sqrt(var + 1e-5) * ln_g_ref[...] + ln_b_ref[...]
    comb = jnp.concatenate([xs, ns], axis=1)
    pre = _nt(comb, w1_ref[...]) + b1_ref[...]
    h1 = 0.5 * pre * (1.0 + lax.erf(pre * (1.0 / math.sqrt(2.0))))
    h1t = h1.astype(jnp.bfloat16).astype(jnp.float32)
    w2t = w2_ref[...].astype(jnp.bfloat16).astype(jnp.float32)
    dlt = jax.nn.sigmoid(
        jnp.sum(h1t * w2t, axis=1, keepdims=True) + b2_ref[0, 0])
    na = jnp.clip(0.7 * acts + 0.3 * dlt, 0.0, 1.0)
    maskq = acts > 0.01
    act_out_ref[...] = jnp.where(maskq, na, 0.0)
    hid_out_ref[...] = jnp.where(maskq, ns, 0.0)


def _topk_kernel(sk_ref, act_full_ref, act_in_ref, sp_act_ref, keep_ref):
    af = act_full_ref[...]
    keys = lax.bitcast_convert_type(af, jnp.int32)
    kcount = jnp.clip(sk_ref[0, 0], 0, KK)

    def cnt(t):
        return jnp.sum((keys >= t).astype(jnp.int32))

    def body(_, carry):
        lo, hi = carry
        mid = (lo + hi) // 2
        big = cnt(mid) >= kcount
        return jnp.where(big, mid, lo), jnp.where(big, hi, mid)

    lo, hi = lax.fori_loop(0, 31, body,
                           (jnp.int32(0), jnp.int32(0x3F800001)))
    t = lo
    gt = keys > t
    eq = keys == t
    c_gt = jnp.sum(gt.astype(jnp.int32))
    need = (kcount - c_gt).astype(jnp.float32)
    eqf = eq.astype(jnp.float32)
    iu = lax.broadcasted_iota(jnp.int32, (128, 128), 0)
    il = lax.broadcasted_iota(jnp.int32, (128, 128), 1)
    upper = (iu < il).astype(jnp.float32)
    within = jnp.dot(eqf, upper, preferred_element_type=jnp.float32)
    rowsum = jnp.sum(eqf, axis=1, keepdims=True)
    ir = lax.broadcasted_iota(jnp.int32, (32, 32), 0)
    ic = lax.broadcasted_iota(jnp.int32, (32, 32), 1)
    lowtri = (ic < ir).astype(jnp.float32)
    rowpre = jnp.dot(lowtri, rowsum, preferred_element_type=jnp.float32)
    rank = within + rowpre
    admit = gt | (eq & (rank < need))
    act_in = act_in_ref[...]
    aa = jnp.sum((act_in > 0.01).astype(jnp.int32)) > 0
    sp = jnp.where(admit, af, 0.0)
    sp_act_ref[...] = jnp.where(aa, sp, act_in)
    keep_ref[...] = jnp.where(aa, admit.astype(jnp.float32), 2.0)


def _select_kernel(keep_ref, hid_full_ref, hid_in_ref, out_ref):
    kp = keep_ref[...]
    out_ref[...] = jnp.where(kp == 2.0, hid_in_ref[...],
                             jnp.where(kp == 1.0, hid_full_ref[...], 0.0))


def kernel(activation, hidden_state, sparsity_k, in_proj_w, in_proj_b,
           out_proj_w, out_proj_b, gru_w_ih, gru_w_hh, gru_b_ih, gru_b_hh,
           ln_g, ln_b, act_w1, act_b1, act_w2, act_b2):
    act_key = activation.reshape(1, N)
    act_col = activation.reshape(N, 1)
    b_in = in_proj_b.reshape(1, D3)
    b_hh = gru_b_hh.reshape(1, D3)
    b_ih = gru_b_ih.reshape(1, D3)
    b_out = out_proj_b.reshape(1, D)
    g2 = ln_g.reshape(1, D)
    bb2 = ln_b.reshape(1, D)
    b1 = act_b1.reshape(1, D)
    b2 = act_b2.reshape(1, 1)
    sk = jnp.asarray(sparsity_k, jnp.int32).reshape(1, 1)

    full = lambda shape: pl.BlockSpec(shape, lambda i: (0,) * len(shape))
    rows = lambda shape: pl.BlockSpec(shape, lambda i: (i, 0))

    q, k, v, gh = pl.pallas_call(
        _proj_kernel,
        grid=(N // BLK_A,),
        in_specs=[rows((BLK_A, D)), full((D3, D)), full((1, D3)),
                  full((D3, D)), full((1, D3))],
        out_specs=[rows((BLK_A, D)), rows((BLK_A, D)), rows((BLK_A, D)),
                   rows((BLK_A, D3))],
        out_shape=[jax.ShapeDtypeStruct((N, D), jnp.float32),
                   jax.ShapeDtypeStruct((N, D), jnp.float32),
                   jax.ShapeDtypeStruct((N, D), jnp.float32),
                   jax.ShapeDtypeStruct((N, D3), jnp.float32)],
    )(hidden_state, in_proj_w, b_in, gru_w_hh, b_hh)

    act_full, hid_full = pl.pallas_call(
        _attn_kernel,
        grid=(N // BLK_Q,),
        in_specs=[full((1, N)), rows((BLK_Q, 1)), rows((BLK_Q, D)),
                  rows((BLK_Q, D)), full((N, D)), full((N, D)),
                  rows((BLK_Q, D3)),
                  full((D, D)), full((1, D)), full((D3, D)), full((1, D3)),
                  full((1, D)), full((1, D)), full((D, 2 * D)), full((1, D)),
                  full((1, D)), pl.BlockSpec(memory_space=pltpu.SMEM)],
        out_specs=[rows((BLK_Q, 1)), rows((BLK_Q, D))],
        out_shape=[jax.ShapeDtypeStruct((N, 1), jnp.float32),
                   jax.ShapeDtypeStruct((N, D), jnp.float32)],
    )(act_key, act_col, hidden_state, q, k, v, gh,
      out_proj_w, b_out, gru_w_ih, b_ih, g2, bb2, act_w1, b1, act_w2, b2)

    sp_act32, keep32 = pl.pallas_call(
        _topk_kernel,
        in_specs=[pl.BlockSpec(memory_space=pltpu.SMEM),
                  pl.BlockSpec((32, 128), lambda: (0, 0)),
                  pl.BlockSpec((32, 128), lambda: (0, 0))],
        out_specs=[pl.BlockSpec((32, 128), lambda: (0, 0)),
                   pl.BlockSpec((32, 128), lambda: (0, 0))],
        out_shape=[jax.ShapeDtypeStruct((32, 128), jnp.float32),
                   jax.ShapeDtypeStruct((32, 128), jnp.float32)],
    )(sk, act_full.reshape(32, 128), activation.reshape(32, 128))

    sp_hid = pl.pallas_call(
        _select_kernel,
        grid=(N // BLK_D,),
        in_specs=[rows((BLK_D, 1)), rows((BLK_D, D)), rows((BLK_D, D))],
        out_specs=rows((BLK_D, D)),
        out_shape=jax.ShapeDtypeStruct((N, D), jnp.float32),
    )(keep32.reshape(N, 1), hid_full, hidden_state)

    return sp_act32.reshape(N), sp_hid


# speedup vs baseline: 1.8873x; 1.0900x over previous
import math

import jax
import jax.numpy as jnp
from jax import lax
from jax.experimental import pallas as pl
from jax.experimental.pallas import tpu as pltpu

N = 4096
D = 256
H = 4
DH = D // H
D3 = 3 * D

BLK_A = 512
BLK_Q = 512
BLK_D = 512
KK = 256

_NEG_INF = float("-inf")


def _nt(a, b):
    return lax.dot_general(a, b, (((1,), (1,)), ((), ())),
                           preferred_element_type=jnp.float32)


def _proj_kernel(x_ref, w_in_ref, b_in_ref, w_hh_ref, b_hh_ref,
                 q_ref, k_ref, v_ref, gh_ref):
    xs = x_ref[...]
    qkv = _nt(xs, w_in_ref[...]) + b_in_ref[...]
    q_ref[...] = qkv[:, :D]
    k_ref[...] = qkv[:, D:2 * D]
    v_ref[...] = qkv[:, 2 * D:]
    gh_ref[...] = _nt(xs, w_hh_ref[...]) + b_hh_ref[...]


def _attn_kernel(act_key_ref, act_col_ref, x_ref, q_ref, k_ref, v_ref, gh_ref,
                 w_out_ref, b_out_ref, w_ih_ref, b_ih_ref,
                 ln_g_ref, ln_b_ref, w1_ref, b1_ref, w2_ref, b2_ref,
                 act_out_ref, hid_out_ref):
    xs = x_ref[...]
    q = q_ref[...]
    kk = k_ref[...]
    vv = v_ref[...]
    maskk = act_key_ref[...] > 0.01
    scale = 1.0 / math.sqrt(DH)
    nq = q.shape[0]
    outs = []
    for h in range(H):
        qh = q[:, h * DH:(h + 1) * DH] * scale
        kh = kk[:, h * DH:(h + 1) * DH]
        vh = vv[:, h * DH:(h + 1) * DH]
        s_full = _nt(qh, kh)
        s_full = jnp.where(maskk, s_full, _NEG_INF)
        KC = 1024
        m_old = jnp.full((nq, 1), _NEG_INF, jnp.float32)
        l_old = jnp.zeros((nq, 1), jnp.float32)
        o_old = jnp.zeros((nq, DH), jnp.float32)
        for j in range(N // KC):
            s = s_full[:, j * KC:(j + 1) * KC]
            mb = jnp.max(s, axis=1, keepdims=True)
            m_new = jnp.maximum(m_old, mb)
            corr = jnp.where(m_old == m_new, 0.0, m_old - m_new)
            e = jnp.exp(s - m_new)
            ec = jnp.exp(corr)
            l_new = ec * l_old + jnp.sum(e, axis=1, keepdims=True)
            acc = (ec * l_old) * o_old
            onum = jnp.dot(e, vh[j * KC:(j + 1) * KC, :],
                           preferred_element_type=jnp.float32) + acc
            o_old = onum * (1.0 / l_new)
            m_old, l_old = m_new, l_new
        outs.append(o_old)
    o = jnp.concatenate(outs, axis=1)
    attn = _nt(o, w_out_ref[...]) + b_out_ref[...]
    acts = act_col_ref[...]
    msg = attn * acts
    gi = _nt(msg, w_ih_ref[...]) + b_ih_ref[...]
    gh = gh_ref[...]
    r = jax.nn.sigmoid(gi[:, :D] + gh[:, :D])
    z = jax.nn.sigmoid(gi[:, D:2 * D] + gh[:, D:2 * D])
    n = jnp.tanh(gi[:, 2 * D:] + r * gh[:, 2 * D:])
    ns = (1.0 - z) * n + z * xs
    mu = jnp.mean(ns, axis=1, keepdims=True)
    var = jnp.mean((ns - mu) ** 2, axis=1, keepdims=True)
    ns = (ns - mu) / jnp.sqrt(var + 1e-5) * ln_g_ref[...] + ln_b_ref[...]
    comb = jnp.concatenate([xs, ns], axis=1)
    pre = _nt(comb, w1_ref[...]) + b1_ref[...]
    h1 = 0.5 * pre * (1.0 + lax.erf(pre * (1.0 / math.sqrt(2.0))))
    h1t = h1.astype(jnp.bfloat16).astype(jnp.float32)
    w2t = w2_ref[...].astype(jnp.bfloat16).astype(jnp.float32)
    dlt = jax.nn.sigmoid(
        jnp.sum(h1t * w2t, axis=1, keepdims=True) + b2_ref[0, 0])
    na = jnp.clip(0.7 * acts + 0.3 * dlt, 0.0, 1.0)
    maskq = acts > 0.01
    act_out_ref[...] = jnp.where(maskq, na, 0.0)
    hid_out_ref[...] = jnp.where(maskq, ns, 0.0)


def _topk_kernel(sk_ref, act_full_ref, act_in_ref, sp_act_ref, keep_ref):
    af = act_full_ref[...]
    keys = lax.bitcast_convert_type(af, jnp.int32)
    kcount = jnp.clip(sk_ref[0, 0], 0, KK)

    def cnt(t):
        return jnp.sum((keys >= t).astype(jnp.int32))

    def body(_, carry):
        lo, hi = carry
        mid = (lo + hi) // 2
        big = cnt(mid) >= kcount
        return jnp.where(big, mid, lo), jnp.where(big, hi, mid)

    lo, hi = lax.fori_loop(0, 31, body,
                           (jnp.int32(0), jnp.int32(0x3F800001)))
    t = lo
    gt = keys > t
    eq = keys == t
    c_gt = jnp.sum(gt.astype(jnp.int32))
    need = (kcount - c_gt).astype(jnp.float32)
    eqf = eq.astype(jnp.float32)
    iu = lax.broadcasted_iota(jnp.int32, (128, 128), 0)
    il = lax.broadcasted_iota(jnp.int32, (128, 128), 1)
    upper = (iu < il).astype(jnp.float32)
    within = jnp.dot(eqf, upper, preferred_element_type=jnp.float32)
    rowsum = jnp.sum(eqf, axis=1, keepdims=True)
    ir = lax.broadcasted_iota(jnp.int32, (32, 32), 0)
    ic = lax.broadcasted_iota(jnp.int32, (32, 32), 1)
    lowtri = (ic < ir).astype(jnp.float32)
    rowpre = jnp.dot(lowtri, rowsum, preferred_element_type=jnp.float32)
    rank = within + rowpre
    admit = gt | (eq & (rank < need))
    act_in = act_in_ref[...]
    aa = jnp.sum((act_in > 0.01).astype(jnp.int32)) > 0
    sp = jnp.where(admit, af, 0.0)
    sp_act_ref[...] = jnp.where(aa, sp, act_in)
    keep_ref[...] = jnp.where(aa, admit.astype(jnp.float32), 2.0)


def _select_kernel(keep_ref, hid_full_ref, hid_in_ref, out_ref):
    kp = keep_ref[...]
    out_ref[...] = jnp.where(kp == 2.0, hid_in_ref[...],
                             jnp.where(kp == 1.0, hid_full_ref[...], 0.0))


def kernel(activation, hidden_state, sparsity_k, in_proj_w, in_proj_b,
           out_proj_w, out_proj_b, gru_w_ih, gru_w_hh, gru_b_ih, gru_b_hh,
           ln_g, ln_b, act_w1, act_b1, act_w2, act_b2):
    act_key = activation.reshape(1, N)
    act_col = activation.reshape(N, 1)
    b_in = in_proj_b.reshape(1, D3)
    b_hh = gru_b_hh.reshape(1, D3)
    b_ih = gru_b_ih.reshape(1, D3)
    b_out = out_proj_b.reshape(1, D)
    g2 = ln_g.reshape(1, D)
    bb2 = ln_b.reshape(1, D)
    b1 = act_b1.reshape(1, D)
    b2 = act_b2.reshape(1, 1)
    sk = jnp.asarray(sparsity_k, jnp.int32).reshape(1, 1)

    full = lambda shape: pl.BlockSpec(shape, lambda i: (0,) * len(shape))
    rows = lambda shape: pl.BlockSpec(shape, lambda i: (i, 0))

    q, k, v, gh = pl.pallas_call(
        _proj_kernel,
        grid=(N // BLK_A,),
        in_specs=[rows((BLK_A, D)), full((D3, D)), full((1, D3)),
                  full((D3, D)), full((1, D3))],
        out_specs=[rows((BLK_A, D)), rows((BLK_A, D)), rows((BLK_A, D)),
                   rows((BLK_A, D3))],
        out_shape=[jax.ShapeDtypeStruct((N, D), jnp.float32),
                   jax.ShapeDtypeStruct((N, D), jnp.float32),
                   jax.ShapeDtypeStruct((N, D), jnp.float32),
                   jax.ShapeDtypeStruct((N, D3), jnp.float32)],
    )(hidden_state, in_proj_w, b_in, gru_w_hh, b_hh)

    act_full, hid_full = pl.pallas_call(
        _attn_kernel,
        grid=(N // BLK_Q,),
        in_specs=[full((1, N)), rows((BLK_Q, 1)), rows((BLK_Q, D)),
                  rows((BLK_Q, D)), full((N, D)), full((N, D)),
                  rows((BLK_Q, D3)),
                  full((D, D)), full((1, D)), full((D3, D)), full((1, D3)),
                  full((1, D)), full((1, D)), full((D, 2 * D)), full((1, D)),
                  full((1, D)), pl.BlockSpec(memory_space=pltpu.SMEM)],
        out_specs=[rows((BLK_Q, 1)), rows((BLK_Q, D))],
        out_shape=[jax.ShapeDtypeStruct((N, 1), jnp.float32),
                   jax.ShapeDtypeStruct((N, D), jnp.float32)],
    )(act_key, act_col, hidden_state, q, k, v, gh,
      out_proj_w, b_out, gru_w_ih, b_ih, g2, bb2, act_w1, b1, act_w2, b2)

    sp_act32, keep32 = pl.pallas_call(
        _topk_kernel,
        in_specs=[pl.BlockSpec(memory_space=pltpu.SMEM),
                  pl.BlockSpec((32, 128), lambda: (0, 0)),
                  pl.BlockSpec((32, 128), lambda: (0, 0))],
        out_specs=[pl.BlockSpec((32, 128), lambda: (0, 0)),
                   pl.BlockSpec((32, 128), lambda: (0, 0))],
        out_shape=[jax.ShapeDtypeStruct((32, 128), jnp.float32),
                   jax.ShapeDtypeStruct((32, 128), jnp.float32)],
    )(sk, act_full.reshape(32, 128), activation.reshape(32, 128))

    sp_hid = pl.pallas_call(
        _select_kernel,
        grid=(N // BLK_D,),
        in_specs=[rows((BLK_D, 1)), rows((BLK_D, D)), rows((BLK_D, D))],
        out_specs=rows((BLK_D, D)),
        out_shape=jax.ShapeDtypeStruct((N, D), jnp.float32),
    )(keep32.reshape(N, 1), hid_full, hidden_state)

    return sp_act32.reshape(N), sp_hid


# speedup vs baseline: 1.8986x; 1.0060x over previous
import math

import jax
import jax.numpy as jnp
from jax import lax
from jax.experimental import pallas as pl
from jax.experimental.pallas import tpu as pltpu

N = 4096
D = 256
H = 4
DH = D // H
D3 = 3 * D

BLK_A = 512
BLK_Q = 512
BLK_D = 512
KK = 256

_NEG_INF = float("-inf")


def _nt(a, b):
    return lax.dot_general(a, b, (((1,), (1,)), ((), ())),
                           preferred_element_type=jnp.float32)


def _proj_kernel(x_ref, w_in_ref, b_in_ref, w_hh_ref, b_hh_ref,
                 q_ref, k_ref, v_ref, gh_ref):
    xs = x_ref[...]
    qkv = _nt(xs, w_in_ref[...]) + b_in_ref[...]
    q_ref[...] = qkv[:, :D]
    k_ref[...] = qkv[:, D:2 * D]
    v_ref[...] = qkv[:, 2 * D:]
    gh_ref[...] = _nt(xs, w_hh_ref[...]) + b_hh_ref[...]


def _attn_kernel(act_key_ref, act_col_ref, x_ref, q_ref, k_ref, v_ref, gh_ref,
                 w_out_ref, b_out_ref, w_ih_ref, b_ih_ref,
                 ln_g_ref, ln_b_ref, w1_ref, b1_ref, w2_ref, b2_ref,
                 act_out_ref, hid_out_ref):
    xs = x_ref[...]
    q = q_ref[...]
    kk = k_ref[...]
    vv = v_ref[...]
    maskk = act_key_ref[...] > 0.01
    scale = 1.0 / math.sqrt(DH)
    nq = q.shape[0]
    outs = []
    for h in range(H):
        qh = q[:, h * DH:(h + 1) * DH] * scale
        vh = vv[:, h * DH:(h + 1) * DH]
        KC = 1024
        m_old = jnp.full((nq, 1), _NEG_INF, jnp.float32)
        l_old = jnp.zeros((nq, 1), jnp.float32)
        o_old = jnp.zeros((nq, DH), jnp.float32)
        for j in range(N // KC):
            khj = kk[j * KC:(j + 1) * KC, h * DH:(h + 1) * DH]
            s = _nt(qh, khj)
            s = jnp.where(maskk[:, j * KC:(j + 1) * KC], s, _NEG_INF)
            mb = jnp.max(s, axis=1, keepdims=True)
            m_new = jnp.maximum(m_old, mb)
            corr = jnp.where(m_old == m_new, 0.0, m_old - m_new)
            e = jnp.exp(s - m_new)
            ec = jnp.exp(corr)
            l_new = ec * l_old + jnp.sum(e, axis=1, keepdims=True)
            acc = (ec * l_old) * o_old
            onum = jnp.dot(e, vh[j * KC:(j + 1) * KC, :],
                           preferred_element_type=jnp.float32) + acc
            o_old = onum * (1.0 / l_new)
            m_old, l_old = m_new, l_new
        outs.append(o_old)
    o = jnp.concatenate(outs, axis=1)
    attn = _nt(o, w_out_ref[...]) + b_out_ref[...]
    acts = act_col_ref[...]
    msg = attn * acts
    gi = _nt(msg, w_ih_ref[...]) + b_ih_ref[...]
    gh = gh_ref[...]
    r = jax.nn.sigmoid(gi[:, :D] + gh[:, :D])
    z = jax.nn.sigmoid(gi[:, D:2 * D] + gh[:, D:2 * D])
    n = jnp.tanh(gi[:, 2 * D:] + r * gh[:, 2 * D:])
    ns = (1.0 - z) * n + z * xs
    mu = jnp.mean(ns, axis=1, keepdims=True)
    var = jnp.mean((ns - mu) ** 2, axis=1, keepdims=True)
    ns = (ns - mu) / jnp.sqrt(var + 1e-5) * ln_g_ref[...] + ln_b_ref[...]
    comb = jnp.concatenate([xs, ns], axis=1)
    pre = _nt(comb, w1_ref[...]) + b1_ref[...]
    h1 = 0.5 * pre * (1.0 + lax.erf(pre * (1.0 / math.sqrt(2.0))))
    h1t = h1.astype(jnp.bfloat16).astype(jnp.float32)
    w2t = w2_ref[...].astype(jnp.bfloat16).astype(jnp.float32)
    dlt = jax.nn.sigmoid(
        jnp.sum(h1t * w2t, axis=1, keepdims=True) + b2_ref[0, 0])
    na = jnp.clip(0.7 * acts + 0.3 * dlt, 0.0, 1.0)
    maskq = acts > 0.01
    act_out_ref[...] = jnp.where(maskq, na, 0.0)
    hid_out_ref[...] = jnp.where(maskq, ns, 0.0)


def _topk_kernel(sk_ref, act_full_ref, act_in_ref, sp_act_ref, keep_ref):
    af = act_full_ref[...]
    keys = lax.bitcast_convert_type(af, jnp.int32)
    kcount = jnp.clip(sk_ref[0, 0], 0, KK)

    def cnt(t):
        return jnp.sum((keys >= t).astype(jnp.int32))

    def body(_, carry):
        lo, hi = carry
        mid = (lo + hi) // 2
        big = cnt(mid) >= kcount
        return jnp.where(big, mid, lo), jnp.where(big, hi, mid)

    lo, hi = lax.fori_loop(0, 31, body,
                           (jnp.int32(0), jnp.int32(0x3F800001)))
    t = lo
    gt = keys > t
    eq = keys == t
    c_gt = jnp.sum(gt.astype(jnp.int32))
    need = (kcount - c_gt).astype(jnp.float32)
    eqf = eq.astype(jnp.float32)
    iu = lax.broadcasted_iota(jnp.int32, (128, 128), 0)
    il = lax.broadcasted_iota(jnp.int32, (128, 128), 1)
    upper = (iu < il).astype(jnp.float32)
    within = jnp.dot(eqf, upper, preferred_element_type=jnp.float32)
    rowsum = jnp.sum(eqf, axis=1, keepdims=True)
    ir = lax.broadcasted_iota(jnp.int32, (32, 32), 0)
    ic = lax.broadcasted_iota(jnp.int32, (32, 32), 1)
    lowtri = (ic < ir).astype(jnp.float32)
    rowpre = jnp.dot(lowtri, rowsum, preferred_element_type=jnp.float32)
    rank = within + rowpre
    admit = gt | (eq & (rank < need))
    act_in = act_in_ref[...]
    aa = jnp.sum((act_in > 0.01).astype(jnp.int32)) > 0
    sp = jnp.where(admit, af, 0.0)
    sp_act_ref[...] = jnp.where(aa, sp, act_in)
    keep_ref[...] = jnp.where(aa, admit.astype(jnp.float32), 2.0)


def _select_kernel(keep_ref, hid_full_ref, hid_in_ref, out_ref):
    kp = keep_ref[...]
    out_ref[...] = jnp.where(kp == 2.0, hid_in_ref[...],
                             jnp.where(kp == 1.0, hid_full_ref[...], 0.0))


def kernel(activation, hidden_state, sparsity_k, in_proj_w, in_proj_b,
           out_proj_w, out_proj_b, gru_w_ih, gru_w_hh, gru_b_ih, gru_b_hh,
           ln_g, ln_b, act_w1, act_b1, act_w2, act_b2):
    act_key = activation.reshape(1, N)
    act_col = activation.reshape(N, 1)
    b_in = in_proj_b.reshape(1, D3)
    b_hh = gru_b_hh.reshape(1, D3)
    b_ih = gru_b_ih.reshape(1, D3)
    b_out = out_proj_b.reshape(1, D)
    g2 = ln_g.reshape(1, D)
    bb2 = ln_b.reshape(1, D)
    b1 = act_b1.reshape(1, D)
    b2 = act_b2.reshape(1, 1)
    sk = jnp.asarray(sparsity_k, jnp.int32).reshape(1, 1)

    full = lambda shape: pl.BlockSpec(shape, lambda i: (0,) * len(shape))
    rows = lambda shape: pl.BlockSpec(shape, lambda i: (i, 0))

    q, k, v, gh = pl.pallas_call(
        _proj_kernel,
        grid=(N // BLK_A,),
        in_specs=[rows((BLK_A, D)), full((D3, D)), full((1, D3)),
                  full((D3, D)), full((1, D3))],
        out_specs=[rows((BLK_A, D)), rows((BLK_A, D)), rows((BLK_A, D)),
                   rows((BLK_A, D3))],
        out_shape=[jax.ShapeDtypeStruct((N, D), jnp.float32),
                   jax.ShapeDtypeStruct((N, D), jnp.float32),
                   jax.ShapeDtypeStruct((N, D), jnp.float32),
                   jax.ShapeDtypeStruct((N, D3), jnp.float32)],
    )(hidden_state, in_proj_w, b_in, gru_w_hh, b_hh)

    act_full, hid_full = pl.pallas_call(
        _attn_kernel,
        grid=(N // BLK_Q,),
        in_specs=[full((1, N)), rows((BLK_Q, 1)), rows((BLK_Q, D)),
                  rows((BLK_Q, D)), full((N, D)), full((N, D)),
                  rows((BLK_Q, D3)),
                  full((D, D)), full((1, D)), full((D3, D)), full((1, D3)),
                  full((1, D)), full((1, D)), full((D, 2 * D)), full((1, D)),
                  full((1, D)), pl.BlockSpec(memory_space=pltpu.SMEM)],
        out_specs=[rows((BLK_Q, 1)), rows((BLK_Q, D))],
        out_shape=[jax.ShapeDtypeStruct((N, 1), jnp.float32),
                   jax.ShapeDtypeStruct((N, D), jnp.float32)],
    )(act_key, act_col, hidden_state, q, k, v, gh,
      out_proj_w, b_out, gru_w_ih, b_ih, g2, bb2, act_w1, b1, act_w2, b2)

    sp_act32, keep32 = pl.pallas_call(
        _topk_kernel,
        in_specs=[pl.BlockSpec(memory_space=pltpu.SMEM),
                  pl.BlockSpec((32, 128), lambda: (0, 0)),
                  pl.BlockSpec((32, 128), lambda: (0, 0))],
        out_specs=[pl.BlockSpec((32, 128), lambda: (0, 0)),
                   pl.BlockSpec((32, 128), lambda: (0, 0))],
        out_shape=[jax.ShapeDtypeStruct((32, 128), jnp.float32),
                   jax.ShapeDtypeStruct((32, 128), jnp.float32)],
    )(sk, act_full.reshape(32, 128), activation.reshape(32, 128))

    sp_hid = pl.pallas_call(
        _select_kernel,
        grid=(N // BLK_D,),
        in_specs=[rows((BLK_D, 1)), rows((BLK_D, D)), rows((BLK_D, D))],
        out_specs=rows((BLK_D, D)),
        out_shape=jax.ShapeDtypeStruct((N, D), jnp.float32),
    )(keep32.reshape(N, 1), hid_full, hidden_state)

    return sp_act32.reshape(N), sp_hid


# speedup vs baseline: 1.9169x; 1.0097x over previous
import math

import jax
import jax.numpy as jnp
from jax import lax
from jax.experimental import pallas as pl
from jax.experimental.pallas import tpu as pltpu

N = 4096
D = 256
H = 4
DH = D // H
D3 = 3 * D

BLK_A = 512
BLK_Q = 512
BLK_D = 512
KK = 256

_NEG_INF = float("-inf")


def _nt(a, b):
    return lax.dot_general(a, b, (((1,), (1,)), ((), ())),
                           preferred_element_type=jnp.float32)


def _kv_kernel(x_ref, w_kv_ref, b_kv_ref, k_ref, v_ref):
    xs = x_ref[...]
    kv = _nt(xs, w_kv_ref[...]) + b_kv_ref[...]
    k_ref[...] = kv[:, :D].astype(jnp.bfloat16)
    v_ref[...] = kv[:, D:].astype(jnp.bfloat16)


def _attn_kernel(act_key_ref, act_col_ref, x_ref, k_ref, v_ref,
                 w_q_ref, b_q_ref, w_hh_ref, b_hh_ref,
                 w_out_ref, b_out_ref, w_ih_ref, b_ih_ref,
                 ln_g_ref, ln_b_ref, w1_ref, b1_ref, w2_ref, b2_ref,
                 act_out_ref, hid_out_ref):
    xs = x_ref[...]
    q = _nt(xs, w_q_ref[...]) + b_q_ref[...]
    gh = _nt(xs, w_hh_ref[...]) + b_hh_ref[...]
    kk = k_ref[...]
    vv = v_ref[...]
    maskk = act_key_ref[...] > 0.01
    scale = 1.0 / math.sqrt(DH)
    nq = q.shape[0]
    outs = []
    for h in range(H):
        qh = q[:, h * DH:(h + 1) * DH] * scale
        vh = vv[:, h * DH:(h + 1) * DH]
        KC = 1024
        m_old = jnp.full((nq, 1), _NEG_INF, jnp.float32)
        l_old = jnp.zeros((nq, 1), jnp.float32)
        o_old = jnp.zeros((nq, DH), jnp.float32)
        for j in range(N // KC):
            khj = kk[j * KC:(j + 1) * KC, h * DH:(h + 1) * DH]
            s = _nt(qh, khj)
            s = jnp.where(maskk[:, j * KC:(j + 1) * KC], s, _NEG_INF)
            mb = jnp.max(s, axis=1, keepdims=True)
            m_new = jnp.maximum(m_old, mb)
            corr = jnp.where(m_old == m_new, 0.0, m_old - m_new)
            e = jnp.exp(s - m_new)
            ec = jnp.exp(corr)
            l_new = ec * l_old + jnp.sum(e, axis=1, keepdims=True)
            acc = (ec * l_old) * o_old
            onum = jnp.dot(e, vh[j * KC:(j + 1) * KC, :],
                           preferred_element_type=jnp.float32) + acc
            o_old = onum * (1.0 / l_new)
            m_old, l_old = m_new, l_new
        outs.append(o_old)
    o = jnp.concatenate(outs, axis=1)
    attn = _nt(o, w_out_ref[...]) + b_out_ref[...]
    acts = act_col_ref[...]
    msg = attn * acts
    gi = _nt(msg, w_ih_ref[...]) + b_ih_ref[...]
    r = jax.nn.sigmoid(gi[:, :D] + gh[:, :D])
    z = jax.nn.sigmoid(gi[:, D:2 * D] + gh[:, D:2 * D])
    n = jnp.tanh(gi[:, 2 * D:] + r * gh[:, 2 * D:])
    ns = (1.0 - z) * n + z * xs
    mu = jnp.mean(ns, axis=1, keepdims=True)
    var = jnp.mean((ns - mu) ** 2, axis=1, keepdims=True)
    ns = (ns - mu) / jnp.sqrt(var + 1e-5) * ln_g_ref[...] + ln_b_ref[...]
    comb = jnp.concatenate([xs, ns], axis=1)
    pre = _nt(comb, w1_ref[...]) + b1_ref[...]
    h1 = 0.5 * pre * (1.0 + lax.erf(pre * (1.0 / math.sqrt(2.0))))
    h1t = h1.astype(jnp.bfloat16).astype(jnp.float32)
    w2t = w2_ref[...].astype(jnp.bfloat16).astype(jnp.float32)
    dlt = jax.nn.sigmoid(
        jnp.sum(h1t * w2t, axis=1, keepdims=True) + b2_ref[0, 0])
    na = jnp.clip(0.7 * acts + 0.3 * dlt, 0.0, 1.0)
    maskq = acts > 0.01
    act_out_ref[...] = jnp.where(maskq, na, 0.0)
    hid_out_ref[...] = jnp.where(maskq, ns, 0.0)


def _topk_kernel(sk_ref, act_full_ref, act_in_ref, sp_act_ref, keep_ref):
    af = act_full_ref[...]
    keys = lax.bitcast_convert_type(af, jnp.int32)
    kcount = jnp.clip(sk_ref[0, 0], 0, KK)

    def cnt(t):
        return jnp.sum((keys >= t).astype(jnp.int32))

    def body(_, carry):
        lo, hi = carry
        mid = (lo + hi) // 2
        big = cnt(mid) >= kcount
        return jnp.where(big, mid, lo), jnp.where(big, hi, mid)

    lo, hi = lax.fori_loop(0, 31, body,
                           (jnp.int32(0), jnp.int32(0x3F800001)))
    t = lo
    gt = keys > t
    eq = keys == t
    c_gt = jnp.sum(gt.astype(jnp.int32))
    need = (kcount - c_gt).astype(jnp.float32)
    eqf = eq.astype(jnp.float32)
    iu = lax.broadcasted_iota(jnp.int32, (128, 128), 0)
    il = lax.broadcasted_iota(jnp.int32, (128, 128), 1)
    upper = (iu < il).astype(jnp.float32)
    within = jnp.dot(eqf, upper, preferred_element_type=jnp.float32)
    rowsum = jnp.sum(eqf, axis=1, keepdims=True)
    ir = lax.broadcasted_iota(jnp.int32, (32, 32), 0)
    ic = lax.broadcasted_iota(jnp.int32, (32, 32), 1)
    lowtri = (ic < ir).astype(jnp.float32)
    rowpre = jnp.dot(lowtri, rowsum, preferred_element_type=jnp.float32)
    rank = within + rowpre
    admit = gt | (eq & (rank < need))
    act_in = act_in_ref[...]
    aa = jnp.sum((act_in > 0.01).astype(jnp.int32)) > 0
    sp = jnp.where(admit, af, 0.0)
    sp_act_ref[...] = jnp.where(aa, sp, act_in)
    keep_ref[...] = jnp.where(aa, admit.astype(jnp.float32), 2.0)


def _select_kernel(keep_ref, hid_full_ref, hid_in_ref, out_ref):
    kp = keep_ref[...]
    out_ref[...] = jnp.where(kp == 2.0, hid_in_ref[...],
                             jnp.where(kp == 1.0, hid_full_ref[...], 0.0))


def kernel(activation, hidden_state, sparsity_k, in_proj_w, in_proj_b,
           out_proj_w, out_proj_b, gru_w_ih, gru_w_hh, gru_b_ih, gru_b_hh,
           ln_g, ln_b, act_w1, act_b1, act_w2, act_b2):
    act_key = activation.reshape(1, N)
    act_col = activation.reshape(N, 1)
    w_q = in_proj_w[:D]
    b_q = in_proj_b[:D].reshape(1, D)
    w_kv = in_proj_w[D:]
    b_kv = in_proj_b[D:].reshape(1, 2 * D)
    b_hh = gru_b_hh.reshape(1, D3)
    b_ih = gru_b_ih.reshape(1, D3)
    b_out = out_proj_b.reshape(1, D)
    g2 = ln_g.reshape(1, D)
    bb2 = ln_b.reshape(1, D)
    b1 = act_b1.reshape(1, D)
    b2 = act_b2.reshape(1, 1)
    sk = jnp.asarray(sparsity_k, jnp.int32).reshape(1, 1)

    full = lambda shape: pl.BlockSpec(shape, lambda i: (0,) * len(shape))
    rows = lambda shape: pl.BlockSpec(shape, lambda i: (i, 0))

    k, v = pl.pallas_call(
        _kv_kernel,
        grid=(N // BLK_A,),
        in_specs=[rows((BLK_A, D)), full((2 * D, D)), full((1, 2 * D))],
        out_specs=[rows((BLK_A, D)), rows((BLK_A, D))],
        out_shape=[jax.ShapeDtypeStruct((N, D), jnp.bfloat16),
                   jax.ShapeDtypeStruct((N, D), jnp.bfloat16)],
    )(hidden_state, w_kv, b_kv)

    act_full, hid_full = pl.pallas_call(
        _attn_kernel,
        grid=(N // BLK_Q,),
        in_specs=[full((1, N)), rows((BLK_Q, 1)), rows((BLK_Q, D)),
                  full((N, D)), full((N, D)),
                  full((D, D)), full((1, D)), full((D3, D)), full((1, D3)),
                  full((D, D)), full((1, D)), full((D3, D)), full((1, D3)),
                  full((1, D)), full((1, D)), full((D, 2 * D)), full((1, D)),
                  full((1, D)), pl.BlockSpec(memory_space=pltpu.SMEM)],
        out_specs=[rows((BLK_Q, 1)), rows((BLK_Q, D))],
        out_shape=[jax.ShapeDtypeStruct((N, 1), jnp.float32),
                   jax.ShapeDtypeStruct((N, D), jnp.float32)],
    )(act_key, act_col, hidden_state, k, v,
      w_q, b_q, gru_w_hh, b_hh,
      out_proj_w, b_out, gru_w_ih, b_ih, g2, bb2, act_w1, b1, act_w2, b2)

    sp_act32, keep32 = pl.pallas_call(
        _topk_kernel,
        in_specs=[pl.BlockSpec(memory_space=pltpu.SMEM),
                  pl.BlockSpec((32, 128), lambda: (0, 0)),
                  pl.BlockSpec((32, 128), lambda: (0, 0))],
        out_specs=[pl.BlockSpec((32, 128), lambda: (0, 0)),
                   pl.BlockSpec((32, 128), lambda: (0, 0))],
        out_shape=[jax.ShapeDtypeStruct((32, 128), jnp.float32),
                   jax.ShapeDtypeStruct((32, 128), jnp.float32)],
    )(sk, act_full.reshape(32, 128), activation.reshape(32, 128))

    sp_hid = pl.pallas_call(
        _select_kernel,
        grid=(N // BLK_D,),
        in_specs=[rows((BLK_D, 1)), rows((BLK_D, D)), rows((BLK_D, D))],
        out_specs=rows((BLK_D, D)),
        out_shape=jax.ShapeDtypeStruct((N, D), jnp.float32),
    )(keep32.reshape(N, 1), hid_full, hidden_state)

    return sp_act32.reshape(N), sp_hid


# speedup vs baseline: 1.9889x; 1.0376x over previous
import math

import jax
import jax.numpy as jnp
from jax import lax
from jax.experimental import pallas as pl
from jax.experimental.pallas import tpu as pltpu

N = 4096
D = 256
H = 4
DH = D // H
D3 = 3 * D

BLK_A = 512
BLK_Q = 512
BLK_D = 512
KK = 256

_NEG_INF = float("-inf")


def _nt(a, b):
    return lax.dot_general(a, b, (((1,), (1,)), ((), ())),
                           preferred_element_type=jnp.float32)


def _kv_kernel(x_ref, w_kv_ref, b_kv_ref, k_ref, v_ref):
    xs = x_ref[...]
    kv = _nt(xs, w_kv_ref[...]) + b_kv_ref[...]
    k_ref[...] = kv[:, :D].astype(jnp.bfloat16)
    v_ref[...] = kv[:, D:].astype(jnp.bfloat16)


def _attn_kernel(act_key_ref, act_col_ref, x_ref, k_ref, v_ref,
                 w_q_ref, b_q_ref, w_hh_ref, b_hh_ref,
                 w_out_ref, b_out_ref, w_ih_ref, b_ih_ref,
                 ln_g_ref, ln_b_ref, w1_ref, b1_ref, w2_ref, b2_ref,
                 act_out_ref, hid_out_ref):
    xs = x_ref[...]
    q = _nt(xs, w_q_ref[...]) + b_q_ref[...]
    gh = _nt(xs, w_hh_ref[...]) + b_hh_ref[...]
    kk = k_ref[...]
    vv = v_ref[...]
    maskk = act_key_ref[...] > 0.01
    scale = 1.0 / math.sqrt(DH)
    nq = q.shape[0]
    outs = []
    for h in range(H):
        qh = q[:, h * DH:(h + 1) * DH] * scale
        vh = vv[:, h * DH:(h + 1) * DH]
        KC = 1024
        m_old = jnp.full((nq, 1), _NEG_INF, jnp.float32)
        l_old = jnp.zeros((nq, 1), jnp.float32)
        o_old = jnp.zeros((nq, DH), jnp.float32)
        for j in range(N // KC):
            khj = kk[j * KC:(j + 1) * KC, h * DH:(h + 1) * DH]
            s = _nt(qh, khj)
            s = jnp.where(maskk[:, j * KC:(j + 1) * KC], s, _NEG_INF)
            mb = jnp.max(s, axis=1, keepdims=True)
            m_new = jnp.maximum(m_old, mb)
            corr = jnp.where(m_old == m_new, 0.0, m_old - m_new)
            e = jnp.exp(s - m_new)
            ec = jnp.exp(corr)
            l_new = ec * l_old + jnp.sum(e, axis=1, keepdims=True)
            acc = (ec * l_old) * o_old
            onum = jnp.dot(e, vh[j * KC:(j + 1) * KC, :],
                           preferred_element_type=jnp.float32) + acc
            o_old = onum * (1.0 / l_new)
            m_old, l_old = m_new, l_new
        outs.append(o_old)
    o = jnp.concatenate(outs, axis=1)
    attn = _nt(o, w_out_ref[...]) + b_out_ref[...]
    acts = act_col_ref[...]
    msg = attn * acts
    gi = _nt(msg, w_ih_ref[...]) + b_ih_ref[...]
    r = jax.nn.sigmoid(gi[:, :D] + gh[:, :D])
    z = jax.nn.sigmoid(gi[:, D:2 * D] + gh[:, D:2 * D])
    n = jnp.tanh(gi[:, 2 * D:] + r * gh[:, 2 * D:])
    ns = (1.0 - z) * n + z * xs
    mu = jnp.mean(ns, axis=1, keepdims=True)
    var = jnp.mean((ns - mu) ** 2, axis=1, keepdims=True)
    ns = (ns - mu) / jnp.sqrt(var + 1e-5) * ln_g_ref[...] + ln_b_ref[...]
    comb = jnp.concatenate([xs, ns], axis=1)
    pre = _nt(comb, w1_ref[...]) + b1_ref[...]
    h1 = 0.5 * pre * (1.0 + lax.erf(pre * (1.0 / math.sqrt(2.0))))
    h1t = h1.astype(jnp.bfloat16).astype(jnp.float32)
    w2t = w2_ref[...].astype(jnp.bfloat16).astype(jnp.float32)
    dlt = jax.nn.sigmoid(
        jnp.sum(h1t * w2t, axis=1, keepdims=True) + b2_ref[0, 0])
    na = jnp.clip(0.7 * acts + 0.3 * dlt, 0.0, 1.0)
    maskq = acts > 0.01
    act_out_ref[...] = jnp.where(maskq, na, 0.0)
    hid_out_ref[...] = jnp.where(maskq, ns, 0.0)


def _topk_kernel(sk_ref, act_full_ref, act_in_ref, af_col_ref, hid_full_ref,
                 hid_in_ref, sp_act_ref, sp_hid_ref):
    af = act_full_ref[...]
    keys = lax.bitcast_convert_type(af, jnp.int32)
    kcount = jnp.clip(sk_ref[0, 0], 0, KK)

    def cnt(t):
        return jnp.sum((keys >= t).astype(jnp.int32))

    def body(_, carry):
        lo, hi = carry
        mid = (lo + hi) // 2
        big = cnt(mid) >= kcount
        return jnp.where(big, mid, lo), jnp.where(big, hi, mid)

    lo, hi = lax.fori_loop(0, 31, body,
                           (jnp.int32(0), jnp.int32(0x3F800001)))
    t = lo
    gt = keys > t
    eq = keys == t
    c_gt = jnp.sum(gt.astype(jnp.int32))
    need = kcount - c_gt
    ir = lax.broadcasted_iota(jnp.int32, (32, 128), 0)
    ic = lax.broadcasted_iota(jnp.int32, (32, 128), 1)
    idx = ir * 128 + ic

    def cnt_eq(p):
        return jnp.sum((eq & (idx < p)).astype(jnp.int32))

    def body2(_, carry):
        lo2, hi2 = carry
        mid = (lo2 + hi2) // 2
        small = cnt_eq(mid) >= need
        return jnp.where(small, lo2, mid), jnp.where(small, mid, hi2)

    lo2, p = lax.fori_loop(0, 13, body2, (jnp.int32(0), jnp.int32(N)))
    admit = gt | (eq & (idx < p))
    act_in = act_in_ref[...]
    aa = jnp.sum((act_in > 0.01).astype(jnp.int32)) > 0
    sp = jnp.where(admit, af, 0.0)
    sp_act_ref[...] = jnp.where(aa, sp, act_in)
    keys_col = lax.bitcast_convert_type(af_col_ref[...], jnp.int32)
    idx_col = lax.broadcasted_iota(jnp.int32, (N, 1), 0)
    keep = (keys_col > t) | ((keys_col == t) & (idx_col < p))
    sp_hid_ref[...] = jnp.where(
        aa, jnp.where(keep, hid_full_ref[...], 0.0), hid_in_ref[...])


def kernel(activation, hidden_state, sparsity_k, in_proj_w, in_proj_b,
           out_proj_w, out_proj_b, gru_w_ih, gru_w_hh, gru_b_ih, gru_b_hh,
           ln_g, ln_b, act_w1, act_b1, act_w2, act_b2):
    act_key = activation.reshape(1, N)
    act_col = activation.reshape(N, 1)
    w_q = in_proj_w[:D]
    b_q = in_proj_b[:D].reshape(1, D)
    w_kv = in_proj_w[D:]
    b_kv = in_proj_b[D:].reshape(1, 2 * D)
    b_hh = gru_b_hh.reshape(1, D3)
    b_ih = gru_b_ih.reshape(1, D3)
    b_out = out_proj_b.reshape(1, D)
    g2 = ln_g.reshape(1, D)
    bb2 = ln_b.reshape(1, D)
    b1 = act_b1.reshape(1, D)
    b2 = act_b2.reshape(1, 1)
    sk = jnp.asarray(sparsity_k, jnp.int32).reshape(1, 1)

    full = lambda shape: pl.BlockSpec(shape, lambda i: (0,) * len(shape))
    rows = lambda shape: pl.BlockSpec(shape, lambda i: (i, 0))

    k, v = pl.pallas_call(
        _kv_kernel,
        grid=(N // BLK_A,),
        in_specs=[rows((BLK_A, D)), full((2 * D, D)), full((1, 2 * D))],
        out_specs=[rows((BLK_A, D)), rows((BLK_A, D))],
        out_shape=[jax.ShapeDtypeStruct((N, D), jnp.bfloat16),
                   jax.ShapeDtypeStruct((N, D), jnp.bfloat16)],
    )(hidden_state, w_kv, b_kv)

    act_full, hid_full = pl.pallas_call(
        _attn_kernel,
        grid=(N // BLK_Q,),
        in_specs=[full((1, N)), rows((BLK_Q, 1)), rows((BLK_Q, D)),
                  full((N, D)), full((N, D)),
                  full((D, D)), full((1, D)), full((D3, D)), full((1, D3)),
                  full((D, D)), full((1, D)), full((D3, D)), full((1, D3)),
                  full((1, D)), full((1, D)), full((D, 2 * D)), full((1, D)),
                  full((1, D)), pl.BlockSpec(memory_space=pltpu.SMEM)],
        out_specs=[rows((BLK_Q, 1)), rows((BLK_Q, D))],
        out_shape=[jax.ShapeDtypeStruct((N, 1), jnp.float32),
                   jax.ShapeDtypeStruct((N, D), jnp.float32)],
    )(act_key, act_col, hidden_state, k, v,
      w_q, b_q, gru_w_hh, b_hh,
      out_proj_w, b_out, gru_w_ih, b_ih, g2, bb2, act_w1, b1, act_w2, b2)

    sp_act32, sp_hid = pl.pallas_call(
        _topk_kernel,
        in_specs=[pl.BlockSpec(memory_space=pltpu.SMEM),
                  pl.BlockSpec((32, 128), lambda: (0, 0)),
                  pl.BlockSpec((32, 128), lambda: (0, 0)),
                  pl.BlockSpec((N, 1), lambda: (0, 0)),
                  pl.BlockSpec((N, D), lambda: (0, 0)),
                  pl.BlockSpec((N, D), lambda: (0, 0))],
        out_specs=[pl.BlockSpec((32, 128), lambda: (0, 0)),
                   pl.BlockSpec((N, D), lambda: (0, 0))],
        out_shape=[jax.ShapeDtypeStruct((32, 128), jnp.float32),
                   jax.ShapeDtypeStruct((N, D), jnp.float32)],
    )(sk, act_full.reshape(32, 128), activation.reshape(32, 128),
      act_full, hid_full, hidden_state)

    return sp_act32.reshape(N), sp_hid


# speedup vs baseline: 1.9931x; 1.0021x over previous
import math

import jax
import jax.numpy as jnp
from jax import lax
from jax.experimental import pallas as pl
from jax.experimental.pallas import tpu as pltpu

N = 4096
D = 256
H = 4
DH = D // H
D3 = 3 * D

BLK_A = 512
BLK_Q = 512
BLK_D = 512
KK = 256

_NEG_INF = float("-inf")


def _nt(a, b):
    return lax.dot_general(a, b, (((1,), (1,)), ((), ())),
                           preferred_element_type=jnp.float32)


def _attn_kernel(act_key_ref, act_col_ref, x_ref, w_kv_ref, b_kv_ref,
                 w_q_ref, b_q_ref, w_hh_ref, b_hh_ref,
                 w_out_ref, b_out_ref, w_ih_ref, b_ih_ref,
                 ln_g_ref, ln_b_ref, w1_ref, b1_ref, w2_ref, b2_ref,
                 act_out_ref, hid_out_ref, k_ref, v_ref):
    i = pl.program_id(0)
    nsteps = N // BLK_Q

    @pl.when(i < nsteps)
    def _kv_phase():
        xs = x_ref[...]
        kv = _nt(xs, w_kv_ref[...]) + b_kv_ref[...]
        k_ref[pl.ds(i * BLK_Q, BLK_Q), :] = kv[:, :D].astype(jnp.bfloat16)
        v_ref[pl.ds(i * BLK_Q, BLK_Q), :] = kv[:, D:].astype(jnp.bfloat16)

    @pl.when(i >= nsteps)
    def _attn_phase():
        _attn_body(act_key_ref, act_col_ref, x_ref,
                   w_q_ref, b_q_ref, w_hh_ref, b_hh_ref,
                   w_out_ref, b_out_ref, w_ih_ref, b_ih_ref,
                   ln_g_ref, ln_b_ref, w1_ref, b1_ref, w2_ref, b2_ref,
                   act_out_ref, hid_out_ref, k_ref, v_ref)


def _attn_body(act_key_ref, act_col_ref, x_ref,
               w_q_ref, b_q_ref, w_hh_ref, b_hh_ref,
               w_out_ref, b_out_ref, w_ih_ref, b_ih_ref,
               ln_g_ref, ln_b_ref, w1_ref, b1_ref, w2_ref, b2_ref,
               act_out_ref, hid_out_ref, k_ref, v_ref):
    xs = x_ref[...]
    q = _nt(xs, w_q_ref[...]) + b_q_ref[...]
    gh = _nt(xs, w_hh_ref[...]) + b_hh_ref[...]
    kk = k_ref[...]
    vv = v_ref[...]
    maskk = act_key_ref[...] > 0.01
    scale = 1.0 / math.sqrt(DH)
    nq = q.shape[0]
    outs = []
    for h in range(H):
        qh = q[:, h * DH:(h + 1) * DH] * scale
        vh = vv[:, h * DH:(h + 1) * DH]
        KC = 1024
        m_old = jnp.full((nq, 1), _NEG_INF, jnp.float32)
        l_old = jnp.zeros((nq, 1), jnp.float32)
        o_old = jnp.zeros((nq, DH), jnp.float32)
        for j in range(N // KC):
            khj = kk[j * KC:(j + 1) * KC, h * DH:(h + 1) * DH]
            s = _nt(qh, khj)
            s = jnp.where(maskk[:, j * KC:(j + 1) * KC], s, _NEG_INF)
            mb = jnp.max(s, axis=1, keepdims=True)
            m_new = jnp.maximum(m_old, mb)
            corr = jnp.where(m_old == m_new, 0.0, m_old - m_new)
            e = jnp.exp(s - m_new)
            ec = jnp.exp(corr)
            l_new = ec * l_old + jnp.sum(e, axis=1, keepdims=True)
            acc = (ec * l_old) * o_old
            onum = jnp.dot(e, vh[j * KC:(j + 1) * KC, :],
                           preferred_element_type=jnp.float32) + acc
            o_old = onum * (1.0 / l_new)
            m_old, l_old = m_new, l_new
        outs.append(o_old)
    o = jnp.concatenate(outs, axis=1)
    attn = _nt(o, w_out_ref[...]) + b_out_ref[...]
    acts = act_col_ref[...]
    msg = attn * acts
    gi = _nt(msg, w_ih_ref[...]) + b_ih_ref[...]
    r = jax.nn.sigmoid(gi[:, :D] + gh[:, :D])
    z = jax.nn.sigmoid(gi[:, D:2 * D] + gh[:, D:2 * D])
    n = jnp.tanh(gi[:, 2 * D:] + r * gh[:, 2 * D:])
    ns = (1.0 - z) * n + z * xs
    mu = jnp.mean(ns, axis=1, keepdims=True)
    var = jnp.mean((ns - mu) ** 2, axis=1, keepdims=True)
    ns = (ns - mu) / jnp.sqrt(var + 1e-5) * ln_g_ref[...] + ln_b_ref[...]
    comb = jnp.concatenate([xs, ns], axis=1)
    pre = _nt(comb, w1_ref[...]) + b1_ref[...]
    h1 = 0.5 * pre * (1.0 + lax.erf(pre * (1.0 / math.sqrt(2.0))))
    h1t = h1.astype(jnp.bfloat16).astype(jnp.float32)
    w2t = w2_ref[...].astype(jnp.bfloat16).astype(jnp.float32)
    dlt = jax.nn.sigmoid(
        jnp.sum(h1t * w2t, axis=1, keepdims=True) + b2_ref[0, 0])
    na = jnp.clip(0.7 * acts + 0.3 * dlt, 0.0, 1.0)
    maskq = acts > 0.01
    act_out_ref[...] = jnp.where(maskq, na, 0.0)
    hid_out_ref[...] = jnp.where(maskq, ns, 0.0)


def _topk_kernel(sk_ref, act_full_ref, act_in_ref, af_col_ref, hid_full_ref,
                 hid_in_ref, sp_act_ref, sp_hid_ref):
    af = act_full_ref[...]
    keys = lax.bitcast_convert_type(af, jnp.int32)
    kcount = jnp.clip(sk_ref[0, 0], 0, KK)

    def cnt(t):
        return jnp.sum((keys >= t).astype(jnp.int32))

    def body(_, carry):
        lo, hi = carry
        mid = (lo + hi) // 2
        big = cnt(mid) >= kcount
        return jnp.where(big, mid, lo), jnp.where(big, hi, mid)

    lo, hi = lax.fori_loop(0, 31, body,
                           (jnp.int32(0), jnp.int32(0x3F800001)))
    t = lo
    gt = keys > t
    eq = keys == t
    c_gt = jnp.sum(gt.astype(jnp.int32))
    need = kcount - c_gt
    ir = lax.broadcasted_iota(jnp.int32, (32, 128), 0)
    ic = lax.broadcasted_iota(jnp.int32, (32, 128), 1)
    idx = ir * 128 + ic

    def cnt_eq(p):
        return jnp.sum((eq & (idx < p)).astype(jnp.int32))

    def body2(_, carry):
        lo2, hi2 = carry
        mid = (lo2 + hi2) // 2
        small = cnt_eq(mid) >= need
        return jnp.where(small, lo2, mid), jnp.where(small, mid, hi2)

    lo2, p = lax.fori_loop(0, 13, body2, (jnp.int32(0), jnp.int32(N)))
    admit = gt | (eq & (idx < p))
    act_in = act_in_ref[...]
    aa = jnp.sum((act_in > 0.01).astype(jnp.int32)) > 0
    sp = jnp.where(admit, af, 0.0)
    sp_act_ref[...] = jnp.where(aa, sp, act_in)
    keys_col = lax.bitcast_convert_type(af_col_ref[...], jnp.int32)
    idx_col = lax.broadcasted_iota(jnp.int32, (N, 1), 0)
    keep = (keys_col > t) | ((keys_col == t) & (idx_col < p))
    sp_hid_ref[...] = jnp.where(
        aa, jnp.where(keep, hid_full_ref[...], 0.0), hid_in_ref[...])


def kernel(activation, hidden_state, sparsity_k, in_proj_w, in_proj_b,
           out_proj_w, out_proj_b, gru_w_ih, gru_w_hh, gru_b_ih, gru_b_hh,
           ln_g, ln_b, act_w1, act_b1, act_w2, act_b2):
    act_key = activation.reshape(1, N)
    act_col = activation.reshape(N, 1)
    w_q = in_proj_w[:D]
    b_q = in_proj_b[:D].reshape(1, D)
    w_kv = in_proj_w[D:]
    b_kv = in_proj_b[D:].reshape(1, 2 * D)
    b_hh = gru_b_hh.reshape(1, D3)
    b_ih = gru_b_ih.reshape(1, D3)
    b_out = out_proj_b.reshape(1, D)
    g2 = ln_g.reshape(1, D)
    bb2 = ln_b.reshape(1, D)
    b1 = act_b1.reshape(1, D)
    b2 = act_b2.reshape(1, 1)
    sk = jnp.asarray(sparsity_k, jnp.int32).reshape(1, 1)

    full = lambda shape: pl.BlockSpec(shape, lambda i: (0,) * len(shape))
    rows = lambda shape: pl.BlockSpec(shape, lambda i: (i, 0))

    nsteps = N // BLK_Q
    rows2 = lambda shape: pl.BlockSpec(
        shape, lambda i: (lax.rem(i, nsteps), 0))
    outrows = lambda shape: pl.BlockSpec(
        shape, lambda i: (lax.max(i - nsteps, 0), 0))

    act_full, hid_full = pl.pallas_call(
        _attn_kernel,
        grid=(2 * nsteps,),
        in_specs=[full((1, N)), rows2((BLK_Q, 1)), rows2((BLK_Q, D)),
                  full((2 * D, D)), full((1, 2 * D)),
                  full((D, D)), full((1, D)), full((D3, D)), full((1, D3)),
                  full((D, D)), full((1, D)), full((D3, D)), full((1, D3)),
                  full((1, D)), full((1, D)), full((D, 2 * D)), full((1, D)),
                  full((1, D)), pl.BlockSpec(memory_space=pltpu.SMEM)],
        out_specs=[outrows((BLK_Q, 1)), outrows((BLK_Q, D))],
        out_shape=[jax.ShapeDtypeStruct((N, 1), jnp.float32),
                   jax.ShapeDtypeStruct((N, D), jnp.float32)],
        scratch_shapes=[pltpu.VMEM((N, D), jnp.bfloat16),
                        pltpu.VMEM((N, D), jnp.bfloat16)],
    )(act_key, act_col, hidden_state, w_kv, b_kv,
      w_q, b_q, gru_w_hh, b_hh,
      out_proj_w, b_out, gru_w_ih, b_ih, g2, bb2, act_w1, b1, act_w2, b2)

    sp_act32, sp_hid = pl.pallas_call(
        _topk_kernel,
        in_specs=[pl.BlockSpec(memory_space=pltpu.SMEM),
                  pl.BlockSpec((32, 128), lambda: (0, 0)),
                  pl.BlockSpec((32, 128), lambda: (0, 0)),
                  pl.BlockSpec((N, 1), lambda: (0, 0)),
                  pl.BlockSpec((N, D), lambda: (0, 0)),
                  pl.BlockSpec((N, D), lambda: (0, 0))],
        out_specs=[pl.BlockSpec((32, 128), lambda: (0, 0)),
                   pl.BlockSpec((N, D), lambda: (0, 0))],
        out_shape=[jax.ShapeDtypeStruct((32, 128), jnp.float32),
                   jax.ShapeDtypeStruct((N, D), jnp.float32)],
    )(sk, act_full.reshape(32, 128), activation.reshape(32, 128),
      act_full, hid_full, hidden_state)

    return sp_act32.reshape(N), sp_hid
